# Initial kernel scaffold; baseline (speedup 1.0000x reference)
import jax
import jax.numpy as jnp
from jax.experimental import pallas as pl


def kernel(loc_datas_p, p_c_p, p_e_p, priors, loc_datas_t, p_c_t, p_e_t):
    raise NotImplementedError("write your pallas kernel here")


# speedup vs baseline: 15.0739x; 15.0739x over previous
import functools

import jax
import jax.numpy as jnp
from jax.experimental import pallas as pl
from jax.experimental.pallas import tpu as pltpu

_NUM_CLASSES = 81
_THRESHOLD = 0.5
_NEGPOS_RATIO = 3
_V0 = 0.1
_V1 = 0.2
_P = 8732
_PP = 9216
_R = _PP // 128


def _body(locp_ref, pcp_ref, pep_ref, pr_ref, tr_ref, lab_ref, pet_ref,
          out_ref, *, num_objs, num_frames):
    f32 = jnp.float32
    pcx = pr_ref[0]
    pcy = pr_ref[1]
    pw = pr_ref[2]
    ph = pr_ref[3]
    px1 = pcx - pw * 0.5
    py1 = pcy - ph * 0.5
    px2 = pcx + pw * 0.5
    py2 = pcy + ph * 0.5
    parea = pw * ph
    row = jax.lax.broadcasted_iota(jnp.int32, (_R, 128), 0)
    lane = jax.lax.broadcasted_iota(jnp.int32, (_R, 128), 1)
    iota = row * 128 + lane
    pad = iota >= _P

    best_ov = jnp.full((_R, 128), -1.0, f32)
    best_idx = jnp.zeros((_R, 128), jnp.int32)
    bp_list = []
    for o in range(num_objs):
        acc = jnp.zeros((_R, 128), f32)
        wsum = f32(0.0)
        for f in range(num_frames):
            w = pet_ref[0, o, f]
            x1 = tr_ref[0, o, f, 0]
            y1 = tr_ref[0, o, f, 1]
            x2 = tr_ref[0, o, f, 2]
            y2 = tr_ref[0, o, f, 3]
            iw = jnp.maximum(jnp.minimum(px2, x2) - jnp.maximum(px1, x1), 0.0)
            ih = jnp.maximum(jnp.minimum(py2, y2) - jnp.maximum(py1, y1), 0.0)
            inter = iw * ih
            ta = (x2 - x1) * (y2 - y1)
            acc = acc + (inter / (ta + parea - inter)) * w
            wsum = wsum + w
        ov = acc / jnp.maximum(wsum, 1.0)
        ov = jnp.where(pad, -1.0, ov)
        m = jnp.max(ov)
        bp = jnp.min(jnp.where(ov == m, iota, jnp.int32(2 ** 30)))
        bp_list.append(bp)
        upd = ov > best_ov
        best_ov = jnp.where(upd, ov, best_ov)
        best_idx = jnp.where(upd, o, best_idx)
    for o in range(num_objs):
        msk = iota == bp_list[o]
        best_ov = jnp.where(msk, 2.0, best_ov)
        best_idx = jnp.where(msk, o, best_idx)

    conf = jnp.zeros((_R, 128), jnp.int32)
    for o in range(num_objs):
        conf = jnp.where(best_idx == o, lab_ref[0, 0, o] + 1, conf)
    conf = jnp.where(best_ov < _THRESHOLD, 0, conf)
    pos = conf > 0
    posf = pos.astype(f32)
    np_i = jnp.sum(pos.astype(jnp.int32))

    loss_l = f32(0.0)
    ex_fs = []
    for f in range(num_frames):
        mx1 = jnp.zeros((_R, 128), f32)
        my1 = jnp.zeros((_R, 128), f32)
        mx2 = jnp.zeros((_R, 128), f32)
        my2 = jnp.zeros((_R, 128), f32)
        exg = jnp.zeros((_R, 128), f32)
        for o in range(num_objs):
            sel = best_idx == o
            mx1 = jnp.where(sel, tr_ref[0, o, f, 0], mx1)
            my1 = jnp.where(sel, tr_ref[0, o, f, 1], my1)
            mx2 = jnp.where(sel, tr_ref[0, o, f, 2], mx2)
            my2 = jnp.where(sel, tr_ref[0, o, f, 3], my2)
            exg = jnp.where(sel, pet_ref[0, o, f], exg)
        ex_f = exg * posf
        ex_fs.append(ex_f)
        mskf = (ex_f > 0).astype(f32)
        gcx = ((mx1 + mx2) * 0.5 - pcx) / (_V0 * pw)
        gcy = ((my1 + my2) * 0.5 - pcy) / (_V0 * ph)
        gw = jnp.log(jnp.maximum((mx2 - mx1) / pw, 1e-8)) * (1.0 / _V1)
        gh = jnp.log(jnp.maximum((my2 - my1) / ph, 1e-8)) * (1.0 / _V1)
        for k, g in enumerate((gcx, gcy, gw, gh)):
            d = locp_ref[0, f, k] - g
            ad = jnp.abs(d)
            sl1 = jnp.where(ad < 1.0, 0.5 * d * d, ad - 0.5)
            loss_l = loss_l + jnp.sum(sl1 * mskf)

    cm = pcp_ref[0, 0]
    for c in range(1, _NUM_CLASSES):
        cm = jnp.maximum(cm, pcp_ref[0, c])
    s = jnp.zeros((_R, 128), f32)
    gath = jnp.zeros((_R, 128), f32)
    for c in range(_NUM_CLASSES):
        xc = pcp_ref[0, c]
        s = s + jnp.exp(xc - cm)
        gath = jnp.where(conf == c, xc, gath)
    lse = jnp.log(s) + cm
    ce = lse - gath

    lm = jnp.where(pos, 0.0, ce)
    lm = jnp.where(pad, -1.0, lm)
    lmb = jax.lax.bitcast_convert_type(lm, jnp.int32)
    k = jnp.minimum(np_i * _NEGPOS_RATIO, _P - 1)
    t = jnp.int32(0)
    for bit in range(30, -1, -1):
        cand = t | jnp.int32(1 << bit)
        cnt = jnp.sum((lmb >= cand).astype(jnp.int32))
        t = jnp.where(cnt >= k, cand, t)
    neg = lmb >= t
    mpc = pos | neg
    mpcf = mpc.astype(f32)
    loss_c = jnp.sum(ce * mpcf)

    loss_e = f32(0.0)
    for f in range(num_frames):
        e0 = pep_ref[0, f, 0]
        e1 = pep_ref[0, f, 1]
        mm = jnp.maximum(e0, e1)
        lsee = mm + jnp.log(jnp.exp(e0 - mm) + jnp.exp(e1 - mm))
        pick = jnp.where(ex_fs[f] > 0.5, e1, e0)
        loss_e = loss_e + jnp.sum((lsee - pick) * mpcf)

    ri = jax.lax.broadcasted_iota(jnp.int32, (8, 128), 0)
    li = jax.lax.broadcasted_iota(jnp.int32, (8, 128), 1)
    r0 = ri == 0
    v = jnp.where(r0 & (li == 0), loss_l, 0.0)
    v = v + jnp.where(r0 & (li == 1), loss_c, 0.0)
    v = v + jnp.where(r0 & (li == 2), loss_e, 0.0)
    v = v + jnp.where(r0 & (li == 3), np_i.astype(f32), 0.0)
    out_ref[0] = v


@jax.jit
def kernel(loc_datas_p, p_c_p, p_e_p, priors, loc_datas_t, p_c_t, p_e_t):
    num, num_frames, P, _ = loc_datas_p.shape
    num_objs = p_c_t.shape[1]
    padn = _PP - P
    locp = jnp.pad(loc_datas_p, ((0, 0), (0, 0), (0, padn), (0, 0)))
    locp = jnp.transpose(locp, (0, 1, 3, 2)).reshape(num, num_frames, 4, _R, 128)
    pcp = jnp.pad(p_c_p[:, 0], ((0, 0), (0, padn), (0, 0)))
    pcp = jnp.transpose(pcp, (0, 2, 1)).reshape(num, _NUM_CLASSES, _R, 128)
    pep = jnp.pad(p_e_p, ((0, 0), (0, 0), (0, padn), (0, 0)))
    pep = jnp.transpose(pep, (0, 1, 3, 2)).reshape(num, num_frames, 2, _R, 128)
    pr = jnp.pad(priors, ((0, padn), (0, 0)))
    padmask = (jnp.arange(_PP) >= P)[:, None]
    pr = jnp.where(padmask, jnp.array([0.0, 0.0, 1.0, 1.0], jnp.float32), pr)
    prp = pr.T.reshape(4, _R, 128)
    lab3 = p_c_t.astype(jnp.int32).reshape(num, 1, num_objs)

    body = functools.partial(_body, num_objs=num_objs, num_frames=num_frames)
    out = pl.pallas_call(
        body,
        grid=(num,),
        in_specs=[
            pl.BlockSpec((1, num_frames, 4, _R, 128),
                         lambda i: (i, 0, 0, 0, 0)),
            pl.BlockSpec((1, _NUM_CLASSES, _R, 128), lambda i: (i, 0, 0, 0)),
            pl.BlockSpec((1, num_frames, 2, _R, 128),
                         lambda i: (i, 0, 0, 0, 0)),
            pl.BlockSpec((4, _R, 128), lambda i: (0, 0, 0)),
            pl.BlockSpec((1, num_objs, num_frames, 4),
                         lambda i: (i, 0, 0, 0), memory_space=pltpu.SMEM),
            pl.BlockSpec((1, 1, num_objs),
                         lambda i: (i, 0, 0), memory_space=pltpu.SMEM),
            pl.BlockSpec((1, num_objs, num_frames),
                         lambda i: (i, 0, 0), memory_space=pltpu.SMEM),
        ],
        out_specs=pl.BlockSpec((1, 8, 128), lambda i: (i, 0, 0)),
        out_shape=jax.ShapeDtypeStruct((num, 8, 128), jnp.float32),
        compiler_params=pltpu.CompilerParams(
            dimension_semantics=("parallel",)),
    )(locp, pcp, pep, prp, loc_datas_t, lab3, p_e_t)

    loss_l = jnp.sum(out[:, 0, 0])
    loss_c = jnp.sum(out[:, 0, 1])
    loss_e = jnp.sum(out[:, 0, 2])
    n = jnp.sum(out[:, 0, 3])
    return (loss_l / n, loss_c / n * num_frames, loss_e / n)
